# baseline (speedup 1.0000x reference)
import jax
import jax.numpy as jnp
from jax.experimental import pallas as pl
from jax.experimental.pallas import tpu as pltpu

_B, _L, _D = 4096, 32, 512
_H = 2 * _D
_N = _B * _L
_STRENGTH = 0.5
_TB = 1024
_S2 = 32.0


def _mlp_block(x_ref, m_ref, w1_ref, b1_ref, w2_ref, b2_ref, o_ref):
    x = x_ref[...]
    h = jnp.dot(x.astype(jnp.float8_e4m3fn), w1_ref[...],
                preferred_element_type=jnp.float32).astype(jnp.bfloat16)
    h = h + b1_ref[...]
    a = h + h * jax.lax.erf(h * 0.7071067811865476)
    d = jnp.dot(a.astype(jnp.float8_e4m3fn), w2_ref[...],
                preferred_element_type=jnp.float32)
    d = d * (1.0 / _S2) + b2_ref[...]
    ones = jnp.full((128, _D), 1.0 / 128.0, dtype=jnp.float8_e4m3fn)
    s = jnp.dot(m_ref[...], ones, preferred_element_type=jnp.float32)
    o_ref[...] = x + d * s


def kernel(latent_states, high_risk_mask, W1, b1, W2, b2):
    x2 = latent_states.reshape(_N, _D)
    mrep = jnp.broadcast_to(
        (high_risk_mask.reshape(_N, 1) * _STRENGTH).astype(jnp.float8_e4m3fn),
        (_N, 128))
    out = pl.pallas_call(
        _mlp_block,
        grid=(_N // _TB,),
        in_specs=[
            pl.BlockSpec((_TB, _D), lambda i: (i, 0)),
            pl.BlockSpec((_TB, 128), lambda i: (i, 0)),
            pl.BlockSpec((_D, _H), lambda i: (0, 0)),
            pl.BlockSpec((1, _H), lambda i: (0, 0)),
            pl.BlockSpec((_H, _D), lambda i: (0, 0)),
            pl.BlockSpec((1, _D), lambda i: (0, 0)),
        ],
        out_specs=pl.BlockSpec((_TB, _D), lambda i: (i, 0)),
        out_shape=jax.ShapeDtypeStruct((_N, _D), jnp.float32),
        compiler_params=pltpu.CompilerParams(
            dimension_semantics=("arbitrary",)),
    )(x2, mrep, W1.astype(jnp.float8_e4m3fn),
      b1.astype(jnp.bfloat16).reshape(1, _H),
      (W2 * (0.5 * _S2)).astype(jnp.float8_e4m3fn), b2.reshape(1, _D))
    return out.reshape(_B, _L, _D)


# speedup vs baseline: 1.0671x; 1.0671x over previous
import jax
import jax.numpy as jnp
from jax.experimental import pallas as pl
from jax.experimental.pallas import tpu as pltpu

_B, _L, _D = 4096, 32, 512
_H = 2 * _D
_N = _B * _L
_STRENGTH = 0.5
_TB = 2048
_S2 = 32.0


def _mlp_block(x_ref, m_ref, w1_ref, b1_ref, w2_ref, b2_ref, o_ref):
    x = x_ref[...]
    h = jnp.dot(x.astype(jnp.float8_e4m3fn), w1_ref[...],
                preferred_element_type=jnp.float32).astype(jnp.bfloat16)
    h = h + b1_ref[...]
    a = h + h * jax.lax.erf(h * 0.7071067811865476)
    d = jnp.dot(a.astype(jnp.float8_e4m3fn), w2_ref[...],
                preferred_element_type=jnp.float32)
    d = d * (1.0 / _S2) + b2_ref[...]
    ones = jnp.full((128, _D), 1.0 / 128.0, dtype=jnp.float8_e4m3fn)
    s = jnp.dot(m_ref[...], ones, preferred_element_type=jnp.float32)
    o_ref[...] = x + d * s


def kernel(latent_states, high_risk_mask, W1, b1, W2, b2):
    x2 = latent_states.reshape(_N, _D)
    mrep = jnp.broadcast_to(
        (high_risk_mask.reshape(_N, 1) * _STRENGTH).astype(jnp.float8_e4m3fn),
        (_N, 128))
    out = pl.pallas_call(
        _mlp_block,
        grid=(_N // _TB,),
        in_specs=[
            pl.BlockSpec((_TB, _D), lambda i: (i, 0)),
            pl.BlockSpec((_TB, 128), lambda i: (i, 0)),
            pl.BlockSpec((_D, _H), lambda i: (0, 0)),
            pl.BlockSpec((1, _H), lambda i: (0, 0)),
            pl.BlockSpec((_H, _D), lambda i: (0, 0)),
            pl.BlockSpec((1, _D), lambda i: (0, 0)),
        ],
        out_specs=pl.BlockSpec((_TB, _D), lambda i: (i, 0)),
        out_shape=jax.ShapeDtypeStruct((_N, _D), jnp.float32),
        compiler_params=pltpu.CompilerParams(
            dimension_semantics=("arbitrary",)),
    )(x2, mrep, W1.astype(jnp.float8_e4m3fn),
      b1.astype(jnp.bfloat16).reshape(1, _H),
      (W2 * (0.5 * _S2)).astype(jnp.float8_e4m3fn), b2.reshape(1, _D))
    return out.reshape(_B, _L, _D)


# speedup vs baseline: 1.0968x; 1.0279x over previous
import jax
import jax.numpy as jnp
from jax.experimental import pallas as pl
from jax.experimental.pallas import tpu as pltpu

_B, _L, _D = 4096, 32, 512
_H = 2 * _D
_N = _B * _L
_STRENGTH = 0.5
_TB = 4096
_S2 = 32.0


def _mlp_block(x_ref, m_ref, w1_ref, b1_ref, w2_ref, b2_ref, o_ref):
    x = x_ref[...]
    h = jnp.dot(x.astype(jnp.float8_e4m3fn), w1_ref[...],
                preferred_element_type=jnp.float32).astype(jnp.bfloat16)
    h = h + b1_ref[...]
    a = h + h * jax.lax.erf(h * 0.7071067811865476)
    d = jnp.dot(a.astype(jnp.float8_e4m3fn), w2_ref[...],
                preferred_element_type=jnp.float32)
    d = d * (1.0 / _S2) + b2_ref[...]
    ones = jnp.full((128, _D), 1.0 / 128.0, dtype=jnp.float8_e4m3fn)
    s = jnp.dot(m_ref[...], ones, preferred_element_type=jnp.float32)
    o_ref[...] = x + d * s


def kernel(latent_states, high_risk_mask, W1, b1, W2, b2):
    x2 = latent_states.reshape(_N, _D)
    mrep = jnp.broadcast_to(
        (high_risk_mask.reshape(_N, 1) * _STRENGTH).astype(jnp.float8_e4m3fn),
        (_N, 128))
    out = pl.pallas_call(
        _mlp_block,
        grid=(_N // _TB,),
        in_specs=[
            pl.BlockSpec((_TB, _D), lambda i: (i, 0)),
            pl.BlockSpec((_TB, 128), lambda i: (i, 0)),
            pl.BlockSpec((_D, _H), lambda i: (0, 0)),
            pl.BlockSpec((1, _H), lambda i: (0, 0)),
            pl.BlockSpec((_H, _D), lambda i: (0, 0)),
            pl.BlockSpec((1, _D), lambda i: (0, 0)),
        ],
        out_specs=pl.BlockSpec((_TB, _D), lambda i: (i, 0)),
        out_shape=jax.ShapeDtypeStruct((_N, _D), jnp.float32),
        compiler_params=pltpu.CompilerParams(
            dimension_semantics=("arbitrary",)),
    )(x2, mrep, W1.astype(jnp.float8_e4m3fn),
      b1.astype(jnp.bfloat16).reshape(1, _H),
      (W2 * (0.5 * _S2)).astype(jnp.float8_e4m3fn), b2.reshape(1, _D))
    return out.reshape(_B, _L, _D)
